# baseline (speedup 1.0000x reference)
import functools

import jax
import jax.numpy as jnp
from jax import lax
from jax.experimental import pallas as pl
from jax.experimental.pallas import tpu as pltpu
from jax.experimental.pallas import tpu_sc as plsc

B, S, H, V, P, T = 4, 2048, 1024, 100000, 2048, 2
EPS = 1e-12
NW = 32
TOK = B * S
TPW = TOK // NW
G = 8
NG = TPW // G
NJ = H // 16


def _lanesum(x):
    lanes = jnp.arange(16, dtype=jnp.int32)
    dnums = lax.GatherDimensionNumbers(
        offset_dims=(), collapsed_slice_dims=(0,), start_index_map=(0,))
    for k in (8, 4, 2, 1):
        perm = lax.gather(x, (lanes ^ k)[:, None], dnums, (1,),
                          mode=lax.GatherScatterMode.PROMISE_IN_BOUNDS)
        x = x + perm
    return x


def _rsqrt16(x):
    yi = jnp.int32(0x5F3759DF) - lax.shift_right_logical(
        lax.bitcast_convert_type(x, jnp.int32), 1)
    y = lax.bitcast_convert_type(yi, jnp.float32)
    for _ in range(3):
        y = y * (jnp.float32(1.5) - jnp.float32(0.5) * x * y * y)
    return y


def _body(ids_hbm, tt_hbm, word_hbm, pos_hbm, type_hbm, gamma_hbm, beta_hbm,
          out_hbm, idsv, ttv, wbuf, pbuf, tbuf, gv, bv, wsem, tsem):
    cid = lax.axis_index("c")
    sid = lax.axis_index("s")
    wid = sid * 2 + cid
    base = wid * TPW
    s0 = lax.rem(wid, jnp.int32(S // TPW)) * TPW

    pltpu.sync_copy(ids_hbm.at[wid], idsv)
    pltpu.sync_copy(tt_hbm.at[wid], ttv)
    pltpu.sync_copy(gamma_hbm, gv)
    pltpu.sync_copy(beta_hbm, bv)

    def group(g, _):
        cw = pltpu.async_copy(word_hbm.at[idsv.at[g]], wbuf, wsem)
        ct = pltpu.async_copy(type_hbm.at[ttv.at[g]], tbuf, tsem)
        pltpu.sync_copy(pos_hbm.at[pl.ds(s0 + g * G, G)], pbuf)
        cw.wait()
        ct.wait()

        def row(r, _):
            def accum(j, c):
                sv, qv = c
                off = pl.ds(j * 16, 16)
                x = wbuf[r, off] + pbuf[r, off] + tbuf[r, off]
                wbuf[r, off] = x
                return (sv + x, qv + x * x)

            zeros = jnp.zeros((16,), jnp.float32)
            sv, qv = lax.fori_loop(0, NJ, accum, (zeros, zeros))
            mean = _lanesum(sv) * jnp.float32(1.0 / H)
            ex2 = _lanesum(qv) * jnp.float32(1.0 / H)
            var = ex2 - mean * mean
            inv = _rsqrt16(var + jnp.float32(EPS))

            def norm(j, _):
                off = pl.ds(j * 16, 16)
                x = wbuf[r, off]
                wbuf[r, off] = (x - mean) * inv * gv[off] + bv[off]
                return 0

            lax.fori_loop(0, NJ, norm, 0)
            return 0

        lax.fori_loop(0, G, row, 0)
        pltpu.sync_copy(wbuf, out_hbm.at[pl.ds(base + g * G, G)])
        return 0

    lax.fori_loop(0, NG, group, 0)


@functools.cache
def _build():
    mesh = plsc.VectorSubcoreMesh(core_axis_name="c", subcore_axis_name="s")
    return pl.kernel(
        _body,
        out_type=jax.ShapeDtypeStruct((TOK, H), jnp.float32),
        mesh=mesh,
        scratch_types=[
            pltpu.VMEM((NG, G), jnp.int32),
            pltpu.VMEM((NG, G), jnp.int32),
            pltpu.VMEM((G, H), jnp.float32),
            pltpu.VMEM((G, H), jnp.float32),
            pltpu.VMEM((G, H), jnp.float32),
            pltpu.VMEM((H,), jnp.float32),
            pltpu.VMEM((H,), jnp.float32),
            pltpu.SemaphoreType.DMA,
            pltpu.SemaphoreType.DMA,
        ],
    )


def kernel(input_ids, token_type_ids, word_emb, pos_emb, type_emb, ln_gamma,
           ln_beta):
    ids3 = input_ids.reshape(NW, NG, G).astype(jnp.int32)
    tt3 = token_type_ids.reshape(NW, NG, G).astype(jnp.int32)
    out = _build()(ids3, tt3, word_emb, pos_emb, type_emb, ln_gamma, ln_beta)
    return out.reshape(B, S, H)


# speedup vs baseline: 1.2714x; 1.2714x over previous
import functools

import jax
import jax.numpy as jnp
from jax import lax
from jax.experimental import pallas as pl
from jax.experimental.pallas import tpu as pltpu
from jax.experimental.pallas import tpu_sc as plsc

B, S, H, V, P, T = 4, 2048, 1024, 100000, 2048, 2
EPS = 1e-12
NW = 32
TOK = B * S
TPW = TOK // NW
G = 8
NG = TPW // G
NJ = H // 16


def _lanesum(x):
    lanes = jnp.arange(16, dtype=jnp.int32)
    dnums = lax.GatherDimensionNumbers(
        offset_dims=(), collapsed_slice_dims=(0,), start_index_map=(0,))
    for k in (8, 4, 2, 1):
        perm = lax.gather(x, (lanes ^ k)[:, None], dnums, (1,),
                          mode=lax.GatherScatterMode.PROMISE_IN_BOUNDS)
        x = x + perm
    return x


def _rsqrt16(x):
    yi = jnp.int32(0x5F3759DF) - lax.shift_right_logical(
        lax.bitcast_convert_type(x, jnp.int32), 1)
    y = lax.bitcast_convert_type(yi, jnp.float32)
    for _ in range(3):
        y = y * (jnp.float32(1.5) - jnp.float32(0.5) * x * y * y)
    return y


def _body(ids_hbm, tt_hbm, word_hbm, pos_hbm, type_hbm, gamma_hbm, beta_hbm,
          out_hbm, idsv, ttv,
          wbuf0, wbuf1, pbuf0, pbuf1, tbuf0, tbuf1, obuf0, obuf1, gv, bv,
          wsem0, wsem1, tsem0, tsem1, psem0, psem1, osem0, osem1):
    cid = lax.axis_index("c")
    sid = lax.axis_index("s")
    wid = sid * 2 + cid
    base = wid * TPW
    s0 = lax.rem(wid, jnp.int32(S // TPW)) * TPW

    pltpu.sync_copy(ids_hbm.at[wid], idsv)
    pltpu.sync_copy(tt_hbm.at[wid], ttv)
    pltpu.sync_copy(gamma_hbm, gv)
    pltpu.sync_copy(beta_hbm, bv)

    slot0 = (wbuf0, pbuf0, tbuf0, obuf0, wsem0, psem0, tsem0, osem0)
    slot1 = (wbuf1, pbuf1, tbuf1, obuf1, wsem1, psem1, tsem1, osem1)

    def fire(g, slot):
        wbuf, pbuf, tbuf, _, wsem, psem, tsem, _ = slot
        pltpu.async_copy(word_hbm.at[idsv.at[g]], wbuf, wsem)
        pltpu.async_copy(type_hbm.at[ttv.at[g]], tbuf, tsem)
        pltpu.async_copy(pos_hbm.at[pl.ds(s0 + g * G, G)], pbuf, psem)

    def process(g, slot):
        wbuf, pbuf, tbuf, obuf, wsem, psem, tsem, osem = slot
        pltpu.make_async_copy(word_hbm.at[idsv.at[g]], wbuf, wsem).wait()
        pltpu.make_async_copy(type_hbm.at[ttv.at[g]], tbuf, tsem).wait()
        pltpu.make_async_copy(
            pos_hbm.at[pl.ds(s0 + g * G, G)], pbuf, psem).wait()

        @pl.when(g >= 2)
        def _():
            pltpu.make_async_copy(
                obuf, out_hbm.at[pl.ds(base, G)], osem).wait()

        def row(r, _):
            def accum(j, c):
                sv, qv = c
                off = pl.ds(j * 16, 16)
                x = wbuf[r, off] + pbuf[r, off] + tbuf[r, off]
                obuf[r, off] = x
                return (sv + x, qv + x * x)

            zeros = jnp.zeros((16,), jnp.float32)
            sv, qv = lax.fori_loop(0, NJ, accum, (zeros, zeros))
            mean = _lanesum(sv) * jnp.float32(1.0 / H)
            ex2 = _lanesum(qv) * jnp.float32(1.0 / H)
            var = ex2 - mean * mean
            inv = _rsqrt16(var + jnp.float32(EPS))

            def norm(j, _):
                off = pl.ds(j * 16, 16)
                x = obuf[r, off]
                obuf[r, off] = (x - mean) * inv * gv[off] + bv[off]
                return 0

            lax.fori_loop(0, NJ, norm, 0)
            return 0

        lax.fori_loop(0, G, row, 0)
        pltpu.async_copy(obuf, out_hbm.at[pl.ds(base + g * G, G)], osem)

    fire(0, slot0)

    def pair(gp, _):
        g0 = 2 * gp
        fire(g0 + 1, slot1)
        process(g0, slot0)

        @pl.when(gp < NG // 2 - 1)
        def _():
            fire(g0 + 2, slot0)

        process(g0 + 1, slot1)
        return 0

    lax.fori_loop(0, NG // 2, pair, 0)

    pltpu.make_async_copy(obuf0, out_hbm.at[pl.ds(base, G)], osem0).wait()
    pltpu.make_async_copy(obuf1, out_hbm.at[pl.ds(base, G)], osem1).wait()


@functools.cache
def _build():
    mesh = plsc.VectorSubcoreMesh(core_axis_name="c", subcore_axis_name="s")
    buf = pltpu.VMEM((G, H), jnp.float32)
    return pl.kernel(
        _body,
        out_type=jax.ShapeDtypeStruct((TOK, H), jnp.float32),
        mesh=mesh,
        scratch_types=[
            pltpu.VMEM((NG, G), jnp.int32),
            pltpu.VMEM((NG, G), jnp.int32),
            buf, buf, buf, buf, buf, buf, buf, buf,
            pltpu.VMEM((H,), jnp.float32),
            pltpu.VMEM((H,), jnp.float32),
        ] + [pltpu.SemaphoreType.DMA] * 8,
    )


def kernel(input_ids, token_type_ids, word_emb, pos_emb, type_emb, ln_gamma,
           ln_beta):
    ids3 = input_ids.reshape(NW, NG, G).astype(jnp.int32)
    tt3 = token_type_ids.reshape(NW, NG, G).astype(jnp.int32)
    out = _build()(ids3, tt3, word_emb, pos_emb, type_emb, ln_gamma, ln_beta)
    return out.reshape(B, S, H)


# speedup vs baseline: 1.2825x; 1.0087x over previous
import functools

import jax
import jax.numpy as jnp
from jax import lax
from jax.experimental import pallas as pl
from jax.experimental.pallas import tpu as pltpu
from jax.experimental.pallas import tpu_sc as plsc

B, S, H, V, P, T = 4, 2048, 1024, 100000, 2048, 2
EPS = 1e-12
NW = 32
TOK = B * S
TPW = TOK // NW
G = 8
NG = TPW // G
NJ = H // 16
UNROLL = 8


def _lanesum(x):
    lanes = jnp.arange(16, dtype=jnp.int32)
    dnums = lax.GatherDimensionNumbers(
        offset_dims=(), collapsed_slice_dims=(0,), start_index_map=(0,))
    for k in (8, 4, 2, 1):
        perm = lax.gather(x, (lanes ^ k)[:, None], dnums, (1,),
                          mode=lax.GatherScatterMode.PROMISE_IN_BOUNDS)
        x = x + perm
    return x


def _rsqrt16(x):
    yi = jnp.int32(0x5F3759DF) - lax.shift_right_logical(
        lax.bitcast_convert_type(x, jnp.int32), 1)
    y = lax.bitcast_convert_type(yi, jnp.float32)
    for _ in range(3):
        y = y * (jnp.float32(1.5) - jnp.float32(0.5) * x * y * y)
    return y


def _body(ids_hbm, tt_hbm, word_hbm, pos_hbm, type_hbm, gamma_hbm, beta_hbm,
          out_hbm, idsv, ttv,
          wbuf0, wbuf1, pbuf0, pbuf1, tbuf0, tbuf1, obuf0, obuf1, gv, bv,
          wsem0, wsem1, tsem0, tsem1, psem0, psem1, osem0, osem1):
    cid = lax.axis_index("c")
    sid = lax.axis_index("s")
    wid = sid * 2 + cid
    base = wid * TPW
    s0 = lax.rem(wid, jnp.int32(S // TPW)) * TPW

    pltpu.sync_copy(ids_hbm.at[wid], idsv)
    pltpu.sync_copy(tt_hbm.at[wid], ttv)
    pltpu.sync_copy(gamma_hbm, gv)
    pltpu.sync_copy(beta_hbm, bv)

    slot0 = (wbuf0, pbuf0, tbuf0, obuf0, wsem0, psem0, tsem0, osem0)
    slot1 = (wbuf1, pbuf1, tbuf1, obuf1, wsem1, psem1, tsem1, osem1)

    def fire(g, slot):
        wbuf, pbuf, tbuf, _, wsem, psem, tsem, _ = slot
        pltpu.async_copy(word_hbm.at[idsv.at[g]], wbuf, wsem)
        pltpu.async_copy(type_hbm.at[ttv.at[g]], tbuf, tsem)
        pltpu.async_copy(pos_hbm.at[pl.ds(s0 + g * G, G)], pbuf, psem)

    def process(g, slot):
        wbuf, pbuf, tbuf, obuf, wsem, psem, tsem, osem = slot
        pltpu.make_async_copy(word_hbm.at[idsv.at[g]], wbuf, wsem).wait()
        pltpu.make_async_copy(type_hbm.at[ttv.at[g]], tbuf, tsem).wait()
        pltpu.make_async_copy(
            pos_hbm.at[pl.ds(s0 + g * G, G)], pbuf, psem).wait()

        @pl.when(g >= 2)
        def _():
            pltpu.make_async_copy(
                obuf, out_hbm.at[pl.ds(base, G)], osem).wait()

        zeros = jnp.zeros((16,), jnp.float32)
        svs, qvs = [], []
        for r in range(G):
            def accum(jj, c, r=r):
                sv, qv = c
                for u in range(UNROLL):
                    off = pl.ds((jj * UNROLL + u) * 16, 16)
                    x = wbuf[r, off] + pbuf[r, off] + tbuf[r, off]
                    obuf[r, off] = x
                    sv = sv + x
                    qv = qv + x * x
                return (sv, qv)

            sv, qv = lax.fori_loop(0, NJ // UNROLL, accum, (zeros, zeros))
            svs.append(sv)
            qvs.append(qv)

        means = [_lanesum(sv) * jnp.float32(1.0 / H) for sv in svs]
        ex2s = [_lanesum(qv) * jnp.float32(1.0 / H) for qv in qvs]
        invs = [_rsqrt16(e - m * m + jnp.float32(EPS))
                for e, m in zip(ex2s, means)]

        def norm(j, carry):
            off = pl.ds(j * 16, 16)
            g_j = gv[off]
            b_j = bv[off]
            for r in range(G):
                x = obuf[r, off]
                obuf[r, off] = (x - means[r]) * (invs[r] * g_j) + b_j
            return carry

        lax.fori_loop(0, NJ, norm, 0)
        pltpu.async_copy(obuf, out_hbm.at[pl.ds(base + g * G, G)], osem)

    fire(0, slot0)

    def pair(gp, _):
        g0 = 2 * gp
        fire(g0 + 1, slot1)
        process(g0, slot0)

        @pl.when(gp < NG // 2 - 1)
        def _():
            fire(g0 + 2, slot0)

        process(g0 + 1, slot1)
        return 0

    lax.fori_loop(0, NG // 2, pair, 0)

    pltpu.make_async_copy(obuf0, out_hbm.at[pl.ds(base, G)], osem0).wait()
    pltpu.make_async_copy(obuf1, out_hbm.at[pl.ds(base, G)], osem1).wait()


@functools.cache
def _build():
    mesh = plsc.VectorSubcoreMesh(core_axis_name="c", subcore_axis_name="s")
    buf = pltpu.VMEM((G, H), jnp.float32)
    return pl.kernel(
        _body,
        out_type=jax.ShapeDtypeStruct((TOK, H), jnp.float32),
        mesh=mesh,
        scratch_types=[
            pltpu.VMEM((NG, G), jnp.int32),
            pltpu.VMEM((NG, G), jnp.int32),
            buf, buf, buf, buf, buf, buf, buf, buf,
            pltpu.VMEM((H,), jnp.float32),
            pltpu.VMEM((H,), jnp.float32),
        ] + [pltpu.SemaphoreType.DMA] * 8,
    )


def kernel(input_ids, token_type_ids, word_emb, pos_emb, type_emb, ln_gamma,
           ln_beta):
    ids3 = input_ids.reshape(NW, NG, G).astype(jnp.int32)
    tt3 = token_type_ids.reshape(NW, NG, G).astype(jnp.int32)
    out = _build()(ids3, tt3, word_emb, pos_emb, type_emb, ln_gamma, ln_beta)
    return out.reshape(B, S, H)


# speedup vs baseline: 1.5995x; 1.2472x over previous
import functools

import jax
import jax.numpy as jnp
from jax import lax
from jax.experimental import pallas as pl
from jax.experimental.pallas import tpu as pltpu
from jax.experimental.pallas import tpu_sc as plsc

B, S, H, V, P, T = 4, 2048, 1024, 100000, 2048, 2
EPS = 1e-12
NW = 32
TOK = B * S
TPW = TOK // NW
G = 16
NG = TPW // G
NJ = H // 16
UNROLL = 4

_DNUMS = lax.GatherDimensionNumbers(
    offset_dims=(), collapsed_slice_dims=(0,), start_index_map=(0,))


def _lanesum(x):
    lanes = jnp.arange(16, dtype=jnp.int32)
    for k in (8, 4, 2, 1):
        perm = lax.gather(x, (lanes ^ k)[:, None], _DNUMS, (1,),
                          mode=lax.GatherScatterMode.PROMISE_IN_BOUNDS)
        x = x + perm
    return x


def _lanebcast(x, r):
    idx = jnp.full((16,), r, dtype=jnp.int32)
    return lax.gather(x, idx[:, None], _DNUMS, (1,),
                      mode=lax.GatherScatterMode.PROMISE_IN_BOUNDS)


def _rsqrt16(x):
    yi = jnp.int32(0x5F3759DF) - lax.shift_right_logical(
        lax.bitcast_convert_type(x, jnp.int32), 1)
    y = lax.bitcast_convert_type(yi, jnp.float32)
    for _ in range(3):
        y = y * (jnp.float32(1.5) - jnp.float32(0.5) * x * y * y)
    return y


def _body(ids_hbm, tt_hbm, word_hbm, pos_hbm, type_hbm, gamma_hbm, beta_hbm,
          out_hbm, idsv, ttv, wbuf0, wbuf1, pbuf0, pbuf1, obuf0, obuf1,
          tv, gv, bv,
          wsem0, wsem1, psem0, psem1, osem0, osem1):
    cid = lax.axis_index("c")
    sid = lax.axis_index("s")
    wid = sid * 2 + cid
    base = wid * TPW
    s0 = lax.rem(wid, jnp.int32(S // TPW)) * TPW

    pltpu.sync_copy(ids_hbm.at[wid], idsv)
    pltpu.sync_copy(tt_hbm.at[wid], ttv)
    pltpu.sync_copy(type_hbm, tv)
    pltpu.sync_copy(gamma_hbm, gv)
    pltpu.sync_copy(beta_hbm, bv)

    slot0 = (wbuf0, pbuf0, obuf0, wsem0, psem0, osem0)
    slot1 = (wbuf1, pbuf1, obuf1, wsem1, psem1, osem1)

    def fire(g, slot):
        wbuf, pbuf, _, wsem, psem, _ = slot
        pltpu.async_copy(word_hbm.at[idsv.at[g]], wbuf, wsem)
        pltpu.async_copy(pos_hbm.at[pl.ds(s0 + g * G, G)], pbuf, psem)

    def process(g, slot):
        wbuf, pbuf, obuf, wsem, psem, osem = slot
        pltpu.make_async_copy(word_hbm.at[idsv.at[g]], wbuf, wsem).wait()
        pltpu.make_async_copy(
            pos_hbm.at[pl.ds(s0 + g * G, G)], pbuf, psem).wait()

        @pl.when(g >= 2)
        def _():
            pltpu.make_async_copy(
                obuf, out_hbm.at[pl.ds(base, G)], osem).wait()

        ttf = ttv[g].astype(jnp.float32)
        tts = [_lanebcast(ttf, r) for r in range(G)]

        zeros = jnp.zeros((16,), jnp.float32)
        svs, qvs = [], []
        for r in range(G):
            def accum(jj, c, r=r):
                sv, qv = c
                for u in range(UNROLL):
                    off = pl.ds((jj * UNROLL + u) * 16, 16)
                    t0 = tv[0, off]
                    t1 = tv[1, off]
                    x = (wbuf[r, off] + pbuf[r, off]) + (
                        t0 + tts[r] * (t1 - t0))
                    obuf[r, off] = x
                    sv = sv + x
                    qv = qv + x * x
                return (sv, qv)

            sv, qv = lax.fori_loop(0, NJ // UNROLL, accum, (zeros, zeros))
            svs.append(sv)
            qvs.append(qv)

        means = [_lanesum(sv) * jnp.float32(1.0 / H) for sv in svs]
        ex2s = [_lanesum(qv) * jnp.float32(1.0 / H) for qv in qvs]
        invs = [_rsqrt16(e - m * m + jnp.float32(EPS))
                for e, m in zip(ex2s, means)]

        for r0 in (0, G // 2):
            def norm(j, carry, r0=r0):
                off = pl.ds(j * 16, 16)
                g_j = gv[off]
                b_j = bv[off]
                for r in range(r0, r0 + G // 2):
                    x = obuf[r, off]
                    obuf[r, off] = (x - means[r]) * (invs[r] * g_j) + b_j
                return carry

            lax.fori_loop(0, NJ, norm, 0)
        pltpu.async_copy(obuf, out_hbm.at[pl.ds(base + g * G, G)], osem)

    fire(0, slot0)

    def pair(gp, _):
        g0 = 2 * gp
        fire(g0 + 1, slot1)
        process(g0, slot0)

        @pl.when(gp < NG // 2 - 1)
        def _():
            fire(g0 + 2, slot0)

        process(g0 + 1, slot1)
        return 0

    lax.fori_loop(0, NG // 2, pair, 0)

    pltpu.make_async_copy(obuf0, out_hbm.at[pl.ds(base, G)], osem0).wait()
    pltpu.make_async_copy(obuf1, out_hbm.at[pl.ds(base, G)], osem1).wait()


@functools.cache
def _build():
    mesh = plsc.VectorSubcoreMesh(core_axis_name="c", subcore_axis_name="s")
    buf = pltpu.VMEM((G, H), jnp.float32)
    return pl.kernel(
        _body,
        out_type=jax.ShapeDtypeStruct((TOK, H), jnp.float32),
        mesh=mesh,
        scratch_types=[
            pltpu.VMEM((NG, G), jnp.int32),
            pltpu.VMEM((NG, G), jnp.int32),
            buf, buf, buf, buf, buf, buf,
            pltpu.VMEM((T, H), jnp.float32),
            pltpu.VMEM((H,), jnp.float32),
            pltpu.VMEM((H,), jnp.float32),
        ] + [pltpu.SemaphoreType.DMA] * 6,
    )


def kernel(input_ids, token_type_ids, word_emb, pos_emb, type_emb, ln_gamma,
           ln_beta):
    ids3 = input_ids.reshape(NW, NG, G).astype(jnp.int32)
    tt3 = token_type_ids.reshape(NW, NG, G).astype(jnp.int32)
    out = _build()(ids3, tt3, word_emb, pos_emb, type_emb, ln_gamma, ln_beta)
    return out.reshape(B, S, H)
